# Initial kernel scaffold; baseline (speedup 1.0000x reference)
import jax
import jax.numpy as jnp
from jax.experimental import pallas as pl


def kernel(actions, obs, eic, eid, eit, batch, w1r, w1n, b1, w2r, w2n, b2, w3r, w3n, b3, w4r, w4n, b4, w5r, w5n, b5, lin2_w, lin2_b, lin3_w, lin3_b, lins0_w, lins0_b, lins1_w, lins1_b, lin1_w, lin1_b):
    raise NotImplementedError("write your pallas kernel here")


# speedup vs baseline: 3.8669x; 3.8669x over previous
import functools

import jax
import jax.numpy as jnp
from jax import lax
from jax.experimental import pallas as pl
from jax.experimental.pallas import tpu as pltpu
from jax.experimental.pallas import tpu_sc as plsc

N = 10000
E = 320000
F = 128
H = 128
A = 256

NC = 2
NS = 16
L = 16
NW = NC * NS

K = 128
EW_CH = -(-E // (NW * K))
CPW = EW_CH
EPAD = NW * CPW * K
NACC = 10112
RPT = NACC // NS

ROWS = 2528
GRID = NACC // ROWS


def _sc_agg_body(y_hbm, srcp_hbm, dstp_hbm, zeros_hbm, out_hbm,
                 src_v, dst_v, rows_v, acc, gsem):
    c = lax.axis_index("c")
    s = lax.axis_index("s")
    wid = s * NC + c
    r0 = s * RPT
    pltpu.sync_copy(zeros_hbm.at[pl.ds(r0, RPT)], acc.at[pl.ds(r0, RPT)])
    pltpu.sync_copy(srcp_hbm.at[wid], src_v)
    pltpu.sync_copy(dstp_hbm.at[wid], dst_v)
    plsc.subcore_barrier()

    def chunk(j, carry):
        pltpu.async_copy(y_hbm.at[src_v.at[j]], rows_v, gsem).wait()
        pltpu.sync_copy(rows_v, acc.at[dst_v.at[j]], add=True)
        return carry

    lax.fori_loop(0, CPW, chunk, 0)
    plsc.subcore_barrier()
    pltpu.sync_copy(acc.at[pl.ds(r0, RPT)], out_hbm.at[c, pl.ds(r0, RPT)])


_sc_agg = functools.partial(
    pl.kernel,
    out_type=jax.ShapeDtypeStruct((NC, NACC, H), jnp.float32),
    mesh=plsc.VectorSubcoreMesh(core_axis_name="c", subcore_axis_name="s"),
    scratch_types=[
        pltpu.VMEM((CPW, K), jnp.int32),
        pltpu.VMEM((CPW, K), jnp.int32),
        pltpu.VMEM((K, H), jnp.float32),
        pltpu.VMEM_SHARED((NACC, H), jnp.float32),
        pltpu.SemaphoreType.DMA,
    ],
)(_sc_agg_body)


NF = 64


def _sc_score_body(x_hbm, idxs_hbm, idxd_hbm, out_hbm,
                   isv, idv, srows, drows, pb, lsh, lall, sem):
    c = lax.axis_index("c")
    s = lax.axis_index("s")

    @pl.when(c == 0)
    def _():
        pltpu.sync_copy(idxs_hbm.at[s], isv)
        pltpu.sync_copy(idxd_hbm.at[s], idv)
        pltpu.async_copy(x_hbm.at[isv], srows, sem).wait()
        pltpu.async_copy(x_hbm.at[idv], drows, sem).wait()

        def acta(a, accs):
            return tuple(
                accs[g] + srows[a, pl.ds(g * L, L)] * drows[a, pl.ds(NF + g * L, L)]
                for g in range(4))

        z = jnp.zeros((L,), jnp.float32)
        accs = lax.fori_loop(0, L, acta, (z, z, z, z))
        for g in range(4):
            pb[pl.ds(g * L, L)] = accs[g]
        pltpu.sync_copy(pb, lsh.at[s])

    plsc.subcore_barrier()

    @pl.when((c == 0) & (s == 0))
    def _():
        pltpu.sync_copy(lsh, lall)

        def rrow(r, accs):
            return tuple(accs[g] + lall[r, pl.ds(g * L, L)] for g in range(4))

        z = jnp.zeros((L,), jnp.float32)
        cols = lax.fori_loop(0, NS, rrow, (z, z, z, z))
        lane = lax.iota(jnp.int32, L)

        def red_all(v, op):
            for sh in (1, 2, 4, 8):
                v = op(v, v[lane ^ sh])
            return v

        mx = jnp.maximum(jnp.maximum(cols[0], cols[1]),
                         jnp.maximum(cols[2], cols[3]))
        mx = red_all(mx, jnp.maximum)
        es = tuple(jnp.exp(cols[g] - mx) for g in range(4))
        tot = red_all(es[0] + es[1] + es[2] + es[3], jnp.add)
        inv = 1.0 / tot
        for g in range(4):
            pb[pl.ds(g * L, L)] = es[g] * inv
        pltpu.sync_copy(pb, out_hbm.at[0])


_sc_score = functools.partial(
    pl.kernel,
    out_type=jax.ShapeDtypeStruct((1, NF), jnp.float32),
    mesh=plsc.VectorSubcoreMesh(core_axis_name="c", subcore_axis_name="s"),
    scratch_types=[
        pltpu.VMEM((L,), jnp.int32),
        pltpu.VMEM((L,), jnp.int32),
        pltpu.VMEM((L, H), jnp.float32),
        pltpu.VMEM((L, H), jnp.float32),
        pltpu.VMEM((NF,), jnp.float32),
        pltpu.VMEM_SHARED((NS, NF), jnp.float32),
        pltpu.VMEM((NS, NF), jnp.float32),
        pltpu.SemaphoreType.DMA,
    ],
)(_sc_score_body)


def _dot(a, b):
    return jnp.dot(a, b, preferred_element_type=jnp.float32)


def _body_first(x_ref, wn_ref, y_ref):
    y_ref[...] = _dot(x_ref[...], wn_ref[...])


def _body_layer(x_ref, agg_ref, wr_ref, b_ref, wn_ref, xo_ref, y_ref):
    h = _dot(x_ref[...], wr_ref[...]) + agg_ref[0] + agg_ref[1] + b_ref[...]
    x = jnp.maximum(h, 0.0)
    xo_ref[...] = x
    y_ref[...] = _dot(x, wn_ref[...])


def _body_layer_dense(x_ref, agg_ref, wr_ref, b_ref, wd_ref, bd_ref,
                      wn_ref, xo_ref, y_ref):
    h = _dot(x_ref[...], wr_ref[...]) + agg_ref[0] + agg_ref[1] + b_ref[...]
    x = jnp.maximum(h, 0.0)
    x = jnp.maximum(_dot(x, wd_ref[...]) + bd_ref[...], 0.0)
    xo_ref[...] = x
    y_ref[...] = _dot(x, wn_ref[...])


def _body_tail(x_ref, agg_ref, wr_ref, b_ref,
               w3_ref, b3_ref, w0_ref, b0_ref, w1_ref, b1_ref,
               wl_ref, bl_ref, xo_ref):
    h = _dot(x_ref[...], wr_ref[...]) + agg_ref[0] + agg_ref[1] + b_ref[...]
    x = jnp.maximum(h, 0.0)
    x = jnp.maximum(_dot(x, w3_ref[...]) + b3_ref[...], 0.0)
    x = _dot(x, w0_ref[...]) + b0_ref[...]
    x = _dot(x, w1_ref[...]) + b1_ref[...]
    xo_ref[...] = _dot(x, wl_ref[...]) + bl_ref[...]


_x_spec = pl.BlockSpec((ROWS, H), lambda i: (i, 0))
_agg_spec = pl.BlockSpec((NC, ROWS, H), lambda i: (0, i, 0))
_w_spec = pl.BlockSpec((H, H), lambda i: (0, 0))
_b_spec = pl.BlockSpec((1, H), lambda i: (0, 0))
_xsd = jax.ShapeDtypeStruct((NACC, H), jnp.float32)


def _call_first(x, wn):
    return pl.pallas_call(
        _body_first,
        grid=(GRID,),
        in_specs=[_x_spec, _w_spec],
        out_specs=_x_spec,
        out_shape=_xsd,
    )(x, wn)


def _call_layer(x, agg, wr, b, wn):
    return pl.pallas_call(
        _body_layer,
        grid=(GRID,),
        in_specs=[_x_spec, _agg_spec, _w_spec, _b_spec, _w_spec],
        out_specs=[_x_spec, _x_spec],
        out_shape=[_xsd, _xsd],
    )(x, agg, wr, b.reshape(1, H), wn)


def _call_layer_dense(x, agg, wr, b, wd, bd, wn):
    return pl.pallas_call(
        _body_layer_dense,
        grid=(GRID,),
        in_specs=[_x_spec, _agg_spec, _w_spec, _b_spec, _w_spec, _b_spec,
                  _w_spec],
        out_specs=[_x_spec, _x_spec],
        out_shape=[_xsd, _xsd],
    )(x, agg, wr, b.reshape(1, H), wd, bd.reshape(1, H), wn)


def _call_tail(x, agg, wr, b, w3, b3, w0, b0, w1, b1, wl, bl):
    return pl.pallas_call(
        _body_tail,
        grid=(GRID,),
        in_specs=[_x_spec, _agg_spec, _w_spec, _b_spec,
                  _w_spec, _b_spec, _w_spec, _b_spec, _w_spec, _b_spec,
                  _w_spec, _b_spec],
        out_specs=_x_spec,
        out_shape=_xsd,
    )(x, agg, wr, b.reshape(1, H),
      w3, b3.reshape(1, H), w0, b0.reshape(1, H), w1, b1.reshape(1, H),
      wl, bl.reshape(1, H))


def _prep_edges(e):
    pad = EPAD - E
    src = jnp.concatenate([e[0], jnp.zeros((pad,), jnp.int32)])
    dst = jnp.concatenate([e[1], jnp.full((pad,), N, jnp.int32)])
    return src.reshape(NW, CPW, K), dst.reshape(NW, CPW, K)


def kernel(actions, obs, eic, eid, eit, batch,
           w1r, w1n, b1, w2r, w2n, b2, w3r, w3n, b3,
           w4r, w4n, b4, w5r, w5n, b5,
           lin2_w, lin2_b, lin3_w, lin3_b,
           lins0_w, lins0_b, lins1_w, lins1_b,
           lin1_w, lin1_b):
    zeros = jnp.zeros((NACC, H), jnp.float32)
    xp = jnp.pad(obs, ((0, NACC - N), (0, 0)))
    sit, dit = _prep_edges(eit)
    sic, dic = _prep_edges(eic)
    sid_, did_ = _prep_edges(eid)

    y = _call_first(xp, w1n)
    agg = _sc_agg(y, sit, dit, zeros)
    x, y = _call_layer(xp, agg, w1r, b1, w2n)
    agg = _sc_agg(y, sic, dic, zeros)
    x, y = _call_layer(x, agg, w2r, b2, w3n)
    agg = _sc_agg(y, sid_, did_, zeros)
    x, y = _call_layer_dense(x, agg, w3r, b3, lin2_w, lin2_b, w3n)
    agg = _sc_agg(y, sit, dit, zeros)
    x, y = _call_layer(x, agg, w3r, b3, w4n)
    agg = _sc_agg(y, sic, dic, zeros)
    x, y = _call_layer(x, agg, w4r, b4, w5n)
    agg = _sc_agg(y, sid_, did_, zeros)
    xfin = _call_tail(x, agg, w5r, b5, lin3_w, lin3_b,
                      lins0_w, lins0_b, lins1_w, lins1_b, lin1_w, lin1_b)

    idx_s = actions[0, :, 0].reshape(NS, L)
    idx_d = actions[0, :, 1].reshape(NS, L)
    probs = _sc_score(xfin, idx_s, idx_d)
    return probs
